# baseline (speedup 1.0000x reference)
import functools
import math

import jax
import jax.numpy as jnp
from jax import lax
from jax.experimental import pallas as pl
from jax.experimental.pallas import tpu as pltpu
from jax.experimental.pallas import tpu_sc as plsc

_BLK = 1024
_H = 256
_CH = 128


def _prep_body(tab_ref, logit_ref, tab2_ref, ls_ref, llcs_ref):
    tab = tab_ref[...]
    tab2_ref[...] = tab + tab
    mx = jnp.max(tab, axis=1, keepdims=True)
    ex = jnp.exp(tab - mx)
    ls_ref[...] = tab - (jnp.log(jnp.sum(ex, axis=1, keepdims=True)) + mx)
    lg = logit_ref[...]
    ml = jnp.max(lg)
    llc = lg - (jnp.log(jnp.sum(jnp.exp(lg - ml))) + ml)
    llcs_ref[...] = llc * (-1.0 / math.log(2.0))


def _vq_body(p_ref, pn_ref, tn_ref, tab_hbm, llcs_hbm,
             idx_ref, bit_ref,
             tab_v, llcs_v, cnt_v, sem):
    i = pl.program_id(0)
    nblk = pl.num_programs(0)
    d = tab_v.shape[0]

    @pl.when(i == 0)
    def _load_tables():
        c1 = pltpu.make_async_copy(tab_hbm, tab_v, sem)
        c2 = pltpu.make_async_copy(llcs_hbm, llcs_v, sem)
        c1.start(); c2.start()
        c1.wait(); c2.wait()
        cnt_v[...] = jnp.zeros((1, d), jnp.float32)

    tn = tn_ref[...]
    pncol = pn_ref[...].reshape(1, _BLK).T
    for h in range(_BLK // _H):
        p = jnp.clip(p_ref[pl.ds(h * _H, _H), :], -1.0, 1.0)
        pn = pncol[h * _H:(h + 1) * _H, :]
        s2 = jax.lax.dot_general(
            p, tab_v[...], (((1,), (1,)), ((), ())),
            preferred_element_type=jnp.float32)
        dist = (tn + pn) - s2
        idx = jnp.argmin(dist, axis=1)
        iota = jax.lax.broadcasted_iota(jnp.int32, dist.shape, 1)
        ohf = (iota == idx[:, None]).astype(jnp.float32)
        cnt_v[...] += jnp.sum(ohf, axis=0, keepdims=True)
        idx_ref[0, 0, pl.ds(h * _H, _H)] = idx

    @pl.when(i == nblk - 1)
    def _finish():
        bit_ref[0, 0] = jnp.sum(cnt_v[...] * llcs_v[...])


def _make_sc_gather(tokens, d, c):
    info = plsc.get_sparse_core_info()
    nw = info.num_cores * info.num_subcores
    bpw = tokens // nw
    mesh = plsc.VectorSubcoreMesh(core_axis_name="c", subcore_axis_name="s")

    @functools.partial(
        pl.kernel, mesh=mesh,
        out_type=[
            jax.ShapeDtypeStruct((tokens, c), jnp.float32),
            jax.ShapeDtypeStruct((tokens, c), jnp.float32),
        ],
        scratch_types=[
            pltpu.VMEM((_CH,), jnp.int32),
            pltpu.VMEM((_CH, c), jnp.float32),
            pltpu.VMEM((_CH, c), jnp.float32),
            pltpu.SemaphoreType.DMA,
            pltpu.SemaphoreType.DMA,
        ],
    )
    def gather(tab_hbm, ls_hbm, idx_hbm, pq_hbm, lpmf_hbm,
               idx_v, ra, rb, sa, sb):
        wid = lax.axis_index("s") * info.num_cores + lax.axis_index("c")
        base = wid * bpw
        for j in range(bpw // _CH):
            off = base + j * _CH
            pltpu.sync_copy(idx_hbm.at[pl.ds(off, _CH)], idx_v)
            ca = pltpu.async_copy(tab_hbm.at[idx_v], ra, sa)
            cb = pltpu.async_copy(ls_hbm.at[idx_v], rb, sb)
            ca.wait()
            cb.wait()
            pltpu.sync_copy(ra, pq_hbm.at[pl.ds(off, _CH)])
            pltpu.sync_copy(rb, lpmf_hbm.at[pl.ds(off, _CH)])

    return gather


def kernel(params, param_table, logits):
    a, b, c = params.shape
    d = param_table.shape[0]
    tokens = a * b
    p2 = params.reshape(tokens, c)
    lg2 = logits.reshape(1, d)
    grid = tokens // _BLK

    pclip = jnp.clip(p2, -1.0, 1.0)
    pn = jnp.sum(pclip ** 2, axis=-1).reshape(grid, 1, _BLK)
    tn = jnp.sum(param_table ** 2, axis=-1).reshape(1, d)

    tab2, ls, llcs = pl.pallas_call(
        _prep_body,
        out_shape=[
            jax.ShapeDtypeStruct((d, c), jnp.float32),
            jax.ShapeDtypeStruct((d, c), jnp.float32),
            jax.ShapeDtypeStruct((1, d), jnp.float32),
        ],
    )(param_table, lg2)

    idx3, bit = pl.pallas_call(
        _vq_body,
        grid=(grid,),
        in_specs=[
            pl.BlockSpec((_BLK, c), lambda i: (i, 0)),
            pl.BlockSpec((1, 1, _BLK), lambda i: (i, 0, 0)),
            pl.BlockSpec((1, d), lambda i: (0, 0)),
            pl.BlockSpec(memory_space=pl.ANY),
            pl.BlockSpec(memory_space=pl.ANY),
        ],
        out_specs=[
            pl.BlockSpec((1, 1, _BLK), lambda i: (i, 0, 0)),
            pl.BlockSpec(memory_space=pltpu.SMEM),
        ],
        out_shape=[
            jax.ShapeDtypeStruct((grid, 1, _BLK), jnp.int32),
            jax.ShapeDtypeStruct((1, 1), jnp.float32),
        ],
        scratch_shapes=[
            pltpu.VMEM((d, c), jnp.float32),
            pltpu.VMEM((1, d), jnp.float32),
            pltpu.VMEM((1, d), jnp.float32),
            pltpu.SemaphoreType.DMA,
        ],
        compiler_params=pltpu.CompilerParams(
            dimension_semantics=("arbitrary",),
        ),
    )(p2, pn, tn, tab2, llcs)

    idx1 = idx3.reshape(tokens)
    pq, lpmf = _make_sc_gather(tokens, d, c)(param_table, ls, idx1)

    return (lpmf.reshape(a, b, c), pq.reshape(a, b, c), bit[0, 0])


# speedup vs baseline: 1.4688x; 1.4688x over previous
import math

import jax
import jax.numpy as jnp
from jax.experimental import pallas as pl
from jax.experimental.pallas import tpu as pltpu

_BLK = 1024
_H = 256


def _prep_body(tab_ref, logit_ref, tab2_ref, gl_ref, llcs_ref):
    c = tab_ref.shape[1]
    tab = tab_ref[...]
    tab2_ref[...] = tab + tab
    mx = jnp.max(tab, axis=1, keepdims=True)
    ex = jnp.exp(tab - mx)
    ls = tab - (jnp.log(jnp.sum(ex, axis=1, keepdims=True)) + mx)
    lg = logit_ref[...]
    ml = jnp.max(lg)
    llc = lg - (jnp.log(jnp.sum(jnp.exp(lg - ml))) + ml)
    llcs_ref[...] = llc * (-1.0 / math.log(2.0))
    gl_ref[:, :c] = tab.astype(jnp.bfloat16)
    gl_ref[:, c:] = ls.astype(jnp.bfloat16)


def _vq_body(p_ref, pn_ref, tn_ref, tab_hbm, gl_hbm, llcs_hbm,
             lpmf_ref, pq_ref, bit_ref,
             tab_v, gl_v, llcs_v, cnt_v, sem):
    i = pl.program_id(0)
    nblk = pl.num_programs(0)
    d = tab_v.shape[0]
    c = tab_v.shape[1]

    @pl.when(i == 0)
    def _load_tables():
        c1 = pltpu.make_async_copy(tab_hbm, tab_v, sem)
        c2 = pltpu.make_async_copy(gl_hbm, gl_v, sem)
        c3 = pltpu.make_async_copy(llcs_hbm, llcs_v, sem)
        c1.start(); c2.start(); c3.start()
        c1.wait(); c2.wait(); c3.wait()
        cnt_v[...] = jnp.zeros((1, d), jnp.float32)

    tn = tn_ref[...]
    pncol = pn_ref[...].reshape(1, _BLK).T
    for h in range(_BLK // _H):
        p = jnp.clip(p_ref[pl.ds(h * _H, _H), :], -1.0, 1.0)
        pn = pncol[h * _H:(h + 1) * _H, :]
        s2 = jax.lax.dot_general(
            p, tab_v[...], (((1,), (1,)), ((), ())),
            preferred_element_type=jnp.float32)
        dist = (tn + pn) - s2
        idx = jnp.argmin(dist, axis=1)
        iota = jax.lax.broadcasted_iota(jnp.int32, dist.shape, 1)
        ohf = (iota == idx[:, None]).astype(jnp.float32)
        cnt_v[...] += jnp.sum(ohf, axis=0, keepdims=True)
        qg = jax.lax.dot_general(
            ohf.astype(jnp.bfloat16), gl_v[...], (((1,), (0,)), ((), ())),
            preferred_element_type=jnp.float32)
        pq_ref[pl.ds(h * _H, _H), :] = qg[:, :c]
        lpmf_ref[pl.ds(h * _H, _H), :] = qg[:, c:]

    @pl.when(i == nblk - 1)
    def _finish():
        bit_ref[0, 0] = jnp.sum(cnt_v[...] * llcs_v[...])


def kernel(params, param_table, logits):
    a, b, c = params.shape
    d = param_table.shape[0]
    tokens = a * b
    p2 = params.reshape(tokens, c)
    lg2 = logits.reshape(1, d)
    grid = tokens // _BLK

    pclip = jnp.clip(p2, -1.0, 1.0)
    pn = jnp.sum(pclip ** 2, axis=-1).reshape(grid, 1, _BLK)
    tn = jnp.sum(param_table ** 2, axis=-1).reshape(1, d)

    tab2, gl, llcs = pl.pallas_call(
        _prep_body,
        out_shape=[
            jax.ShapeDtypeStruct((d, c), jnp.float32),
            jax.ShapeDtypeStruct((d, 2 * c), jnp.bfloat16),
            jax.ShapeDtypeStruct((1, d), jnp.float32),
        ],
    )(param_table, lg2)

    lpmf, pq, bit = pl.pallas_call(
        _vq_body,
        grid=(grid,),
        in_specs=[
            pl.BlockSpec((_BLK, c), lambda i: (i, 0)),
            pl.BlockSpec((1, 1, _BLK), lambda i: (i, 0, 0)),
            pl.BlockSpec((1, d), lambda i: (0, 0)),
            pl.BlockSpec(memory_space=pl.ANY),
            pl.BlockSpec(memory_space=pl.ANY),
            pl.BlockSpec(memory_space=pl.ANY),
        ],
        out_specs=[
            pl.BlockSpec((_BLK, c), lambda i: (i, 0)),
            pl.BlockSpec((_BLK, c), lambda i: (i, 0)),
            pl.BlockSpec(memory_space=pltpu.SMEM),
        ],
        out_shape=[
            jax.ShapeDtypeStruct((tokens, c), jnp.float32),
            jax.ShapeDtypeStruct((tokens, c), jnp.float32),
            jax.ShapeDtypeStruct((1, 1), jnp.float32),
        ],
        scratch_shapes=[
            pltpu.VMEM((d, c), jnp.float32),
            pltpu.VMEM((d, 2 * c), jnp.bfloat16),
            pltpu.VMEM((1, d), jnp.float32),
            pltpu.VMEM((1, d), jnp.float32),
            pltpu.SemaphoreType.DMA,
        ],
        compiler_params=pltpu.CompilerParams(
            dimension_semantics=("arbitrary",),
        ),
    )(p2, pn, tn, tab2, gl, llcs)

    return (lpmf.reshape(a, b, c), pq.reshape(a, b, c), bit[0, 0])
